# initial kernel scaffold (unmeasured)
import jax
import jax.numpy as jnp
from jax import lax
from jax.experimental import pallas as pl
from jax.experimental.pallas import tpu as pltpu

N_DEV = 8


def _gelu(y):
    c = 0.7978845608028654
    return 0.5 * y * (1.0 + jnp.tanh(c * (y + 0.044715 * y * y * y)))


def kernel(x, w_mat):
    m, k_sh = x.shape
    _, n = w_mat.shape
    ch = m // N_DEV

    def body(x_ref, w_ref, out_ref, comm_ref, send_sems, recv_sems):
        my = lax.axis_index("i")
        left = lax.rem(my + N_DEV - 1, N_DEV)
        right = lax.rem(my + 1, N_DEV)

        barrier_sem = pltpu.get_barrier_semaphore()
        for nbr in (left, right):
            pl.semaphore_signal(
                barrier_sem, inc=1,
                device_id=(nbr,), device_id_type=pl.DeviceIdType.MESH,
            )
        pl.semaphore_wait(barrier_sem, 2)

        def partial_chunk(c):
            xs = x_ref[pl.ds(c * ch, ch), :]
            return lax.dot_general(
                xs, w_ref[...],
                (((1,), (0,)), ((), ())),
                preferred_element_type=jnp.float32,
            )

        comm_ref[0] = partial_chunk(my)
        for s in range(N_DEV - 1):
            send_slot = s % 2
            recv_slot = (s + 1) % 2
            rdma = pltpu.make_async_remote_copy(
                src_ref=comm_ref.at[send_slot],
                dst_ref=comm_ref.at[recv_slot],
                send_sem=send_sems.at[send_slot],
                recv_sem=recv_sems.at[recv_slot],
                device_id=(right,),
                device_id_type=pl.DeviceIdType.MESH,
            )
            rdma.start()
            rdma.wait()
            c = lax.rem(my + 2 * N_DEV - s - 1, N_DEV)
            comm_ref[recv_slot] = comm_ref[recv_slot] + partial_chunk(c)

        own = lax.rem(my + 1, N_DEV)
        ge = _gelu(comm_ref[1])
        out_ref[pl.ds(own * ch, ch), :] = ge
        comm_ref[1] = ge

        for t in range(N_DEV - 1):
            s = N_DEV - 1 + t
            send_slot = s % 2
            recv_slot = (s + 1) % 2
            rdma = pltpu.make_async_remote_copy(
                src_ref=comm_ref.at[send_slot],
                dst_ref=comm_ref.at[recv_slot],
                send_sem=send_sems.at[send_slot],
                recv_sem=recv_sems.at[recv_slot],
                device_id=(right,),
                device_id_type=pl.DeviceIdType.MESH,
            )
            rdma.start()
            rdma.wait()
            c = lax.rem(my + 2 * N_DEV - t, N_DEV)
            out_ref[pl.ds(c * ch, ch), :] = comm_ref[recv_slot]

    out_shape = jax.ShapeDtypeStruct((m, n), jnp.float32)
    return pl.pallas_call(
        body,
        out_shape=out_shape,
        in_specs=[
            pl.BlockSpec(memory_space=pltpu.VMEM),
            pl.BlockSpec(memory_space=pltpu.VMEM),
        ],
        out_specs=pl.BlockSpec(memory_space=pltpu.VMEM),
        scratch_shapes=[
            pltpu.VMEM((2, ch, n), jnp.float32),
            pltpu.SemaphoreType.DMA((2,)),
            pltpu.SemaphoreType.DMA((2,)),
        ],
        compiler_params=pltpu.CompilerParams(collective_id=0),
    )(x, w_mat)


# baseline (device time: 718430 ns/iter reference)
import jax
import jax.numpy as jnp
from jax import lax
from jax.experimental import pallas as pl
from jax.experimental.pallas import tpu as pltpu

N_DEV = 8


def _gelu(y):
    c = 0.7978845608028654
    return 0.5 * y * (1.0 + jnp.tanh(c * (y + 0.044715 * y * y * y)))


def kernel(x, w_mat):
    m, k_sh = x.shape
    _, n = w_mat.shape
    ch = m // N_DEV
    x = x.astype(jnp.bfloat16)
    w_mat = w_mat.astype(jnp.bfloat16)

    def body(x_ref, w_ref, out_ref, comm_ref, send_sems, recv_sems):
        my = lax.axis_index("i")
        left = lax.rem(my + N_DEV - 1, N_DEV)
        right = lax.rem(my + 1, N_DEV)

        barrier_sem = pltpu.get_barrier_semaphore()
        for nbr in (left, right):
            pl.semaphore_signal(
                barrier_sem, inc=1,
                device_id=(nbr,), device_id_type=pl.DeviceIdType.MESH,
            )
        pl.semaphore_wait(barrier_sem, 2)

        def partial_chunk(c):
            xs = x_ref[pl.ds(c * ch, ch), :]
            return lax.dot_general(
                xs, w_ref[...],
                (((1,), (0,)), ((), ())),
                preferred_element_type=jnp.float32,
            )

        comm_ref[0] = partial_chunk(my)
        for s in range(N_DEV - 1):
            send_slot = s % 2
            recv_slot = (s + 1) % 2
            rdma = pltpu.make_async_remote_copy(
                src_ref=comm_ref.at[send_slot],
                dst_ref=comm_ref.at[recv_slot],
                send_sem=send_sems.at[send_slot],
                recv_sem=recv_sems.at[recv_slot],
                device_id=(right,),
                device_id_type=pl.DeviceIdType.MESH,
            )
            rdma.start()
            rdma.wait()
            c = lax.rem(my + 2 * N_DEV - s - 1, N_DEV)
            comm_ref[recv_slot] = comm_ref[recv_slot] + partial_chunk(c)

        own = lax.rem(my + 1, N_DEV)
        ge = _gelu(comm_ref[1])
        out_ref[pl.ds(own * ch, ch), :] = ge
        comm_ref[1] = ge

        for t in range(N_DEV - 1):
            s = N_DEV - 1 + t
            send_slot = s % 2
            recv_slot = (s + 1) % 2
            rdma = pltpu.make_async_remote_copy(
                src_ref=comm_ref.at[send_slot],
                dst_ref=comm_ref.at[recv_slot],
                send_sem=send_sems.at[send_slot],
                recv_sem=recv_sems.at[recv_slot],
                device_id=(right,),
                device_id_type=pl.DeviceIdType.MESH,
            )
            rdma.start()
            rdma.wait()
            c = lax.rem(my + 2 * N_DEV - t, N_DEV)
            out_ref[pl.ds(c * ch, ch), :] = comm_ref[recv_slot]

    out_shape = jax.ShapeDtypeStruct((m, n), jnp.float32)
    return pl.pallas_call(
        body,
        out_shape=out_shape,
        in_specs=[
            pl.BlockSpec(memory_space=pltpu.VMEM),
            pl.BlockSpec(memory_space=pltpu.VMEM),
        ],
        out_specs=pl.BlockSpec(memory_space=pltpu.VMEM),
        scratch_shapes=[
            pltpu.VMEM((2, ch, n), jnp.float32),
            pltpu.SemaphoreType.DMA((2,)),
            pltpu.SemaphoreType.DMA((2,)),
        ],
        compiler_params=pltpu.CompilerParams(
            collective_id=0,
            vmem_limit_bytes=60 * 1024 * 1024,
        ),
    )(x, w_mat)


# device time: 403535 ns/iter; 1.7803x vs baseline; 1.7803x over previous
import jax
import jax.numpy as jnp
from jax import lax
from jax.experimental import pallas as pl
from jax.experimental.pallas import tpu as pltpu

N_DEV = 8


def _gelu(y):
    c = 0.7978845608028654
    return 0.5 * y * (1.0 + jnp.tanh(c * (y + 0.044715 * y * y * y)))


def kernel(x, w_mat):
    m, k_sh = x.shape
    _, n = w_mat.shape
    ch = m // N_DEV
    x = x.astype(jnp.bfloat16)
    w_mat = w_mat.astype(jnp.bfloat16)

    def body(x_ref, w_ref, out_ref, comm_ref, send_sems, recv_sems):
        my = lax.axis_index("i")
        left = lax.rem(my + N_DEV - 1, N_DEV)
        right = lax.rem(my + 1, N_DEV)

        barrier_sem = pltpu.get_barrier_semaphore()
        for nbr in (left, right):
            pl.semaphore_signal(
                barrier_sem, inc=1,
                device_id=(nbr,), device_id_type=pl.DeviceIdType.MESH,
            )
        pl.semaphore_wait(barrier_sem, 2)

        def partial_chunk(c):
            xs = x_ref[pl.ds(c * ch, ch), :]
            return lax.dot_general(
                xs, w_ref[...],
                (((1,), (0,)), ((), ())),
                preferred_element_type=jnp.float32,
            )

        comm_ref[0] = partial_chunk(my).astype(jnp.bfloat16)
        for s in range(N_DEV - 1):
            send_slot = s % 2
            recv_slot = (s + 1) % 2
            rdma = pltpu.make_async_remote_copy(
                src_ref=comm_ref.at[send_slot],
                dst_ref=comm_ref.at[recv_slot],
                send_sem=send_sems.at[send_slot],
                recv_sem=recv_sems.at[recv_slot],
                device_id=(right,),
                device_id_type=pl.DeviceIdType.MESH,
            )
            rdma.start()
            rdma.wait()
            c = lax.rem(my + 2 * N_DEV - s - 1, N_DEV)
            comm_ref[recv_slot] = (
                comm_ref[recv_slot].astype(jnp.float32) + partial_chunk(c)
            ).astype(jnp.bfloat16)

        own = lax.rem(my + 1, N_DEV)
        ge = _gelu(comm_ref[1].astype(jnp.float32))
        out_ref[pl.ds(own * ch, ch), :] = ge
        comm_ref[1] = ge.astype(jnp.bfloat16)

        for t in range(N_DEV - 1):
            s = N_DEV - 1 + t
            send_slot = s % 2
            recv_slot = (s + 1) % 2
            rdma = pltpu.make_async_remote_copy(
                src_ref=comm_ref.at[send_slot],
                dst_ref=comm_ref.at[recv_slot],
                send_sem=send_sems.at[send_slot],
                recv_sem=recv_sems.at[recv_slot],
                device_id=(right,),
                device_id_type=pl.DeviceIdType.MESH,
            )
            rdma.start()
            rdma.wait()
            c = lax.rem(my + 2 * N_DEV - t, N_DEV)
            out_ref[pl.ds(c * ch, ch), :] = comm_ref[recv_slot].astype(jnp.float32)

    out_shape = jax.ShapeDtypeStruct((m, n), jnp.float32)
    return pl.pallas_call(
        body,
        out_shape=out_shape,
        in_specs=[
            pl.BlockSpec(memory_space=pltpu.VMEM),
            pl.BlockSpec(memory_space=pltpu.VMEM),
        ],
        out_specs=pl.BlockSpec(memory_space=pltpu.VMEM),
        scratch_shapes=[
            pltpu.VMEM((2, ch, n), jnp.bfloat16),
            pltpu.SemaphoreType.DMA((2,)),
            pltpu.SemaphoreType.DMA((2,)),
        ],
        compiler_params=pltpu.CompilerParams(
            collective_id=0,
            vmem_limit_bytes=60 * 1024 * 1024,
        ),
    )(x, w_mat)


# device time: 251543 ns/iter; 2.8561x vs baseline; 1.6042x over previous
import jax
import jax.numpy as jnp
from jax import lax
from jax.experimental import pallas as pl
from jax.experimental.pallas import tpu as pltpu

N_DEV = 8


def _gelu(y):
    c = 0.7978845608028654
    return 0.5 * y * (1.0 + jnp.tanh(c * (y + 0.044715 * y * y * y)))


def kernel(x, w_mat):
    m, k_sh = x.shape
    _, n = w_mat.shape
    ch = m // N_DEV
    nh = n // 2
    x = x.astype(jnp.bfloat16)
    w_mat = w_mat.astype(jnp.bfloat16)

    def body(x_ref, w_ref, out_ref, comm_r, comm_l, sems):
        my = lax.axis_index("i")
        left = lax.rem(my + N_DEV - 1, N_DEV)
        right = lax.rem(my + 1, N_DEV)

        barrier_sem = pltpu.get_barrier_semaphore()
        for nbr in (left, right):
            pl.semaphore_signal(
                barrier_sem, inc=1,
                device_id=(nbr,), device_id_type=pl.DeviceIdType.MESH,
            )
        pl.semaphore_wait(barrier_sem, 2)

        def partial_chunk(c, half):
            xs = x_ref[pl.ds(c * ch, ch), :]
            ws = w_ref[:, half * nh:(half + 1) * nh]
            return lax.dot_general(
                xs, ws,
                (((1,), (0,)), ((), ())),
                preferred_element_type=jnp.float32,
            )

        def hop(s, comm, direction, dst):
            send_slot = s % 2
            recv_slot = (s + 1) % 2
            return pltpu.make_async_remote_copy(
                src_ref=comm.at[send_slot],
                dst_ref=comm.at[recv_slot],
                send_sem=sems.at[direction, 0, send_slot],
                recv_sem=sems.at[direction, 1, recv_slot],
                device_id=(dst,),
                device_id_type=pl.DeviceIdType.MESH,
            )

        comm_r[0] = partial_chunk(my, 0).astype(jnp.bfloat16)
        comm_l[0] = partial_chunk(my, 1).astype(jnp.bfloat16)
        for s in range(N_DEV - 1):
            recv_slot = (s + 1) % 2
            rr = hop(s, comm_r, 0, right)
            rl = hop(s, comm_l, 1, left)
            rr.start()
            rl.start()
            rr.wait()
            c_r = lax.rem(my + 2 * N_DEV - s - 1, N_DEV)
            comm_r[recv_slot] = (
                comm_r[recv_slot].astype(jnp.float32) + partial_chunk(c_r, 0)
            ).astype(jnp.bfloat16)
            rl.wait()
            c_l = lax.rem(my + s + 1, N_DEV)
            comm_l[recv_slot] = (
                comm_l[recv_slot].astype(jnp.float32) + partial_chunk(c_l, 1)
            ).astype(jnp.bfloat16)

        own_r = lax.rem(my + 1, N_DEV)
        ge_r = _gelu(comm_r[1].astype(jnp.float32))
        out_ref[pl.ds(own_r * ch, ch), :nh] = ge_r
        comm_r[1] = ge_r.astype(jnp.bfloat16)
        own_l = lax.rem(my + N_DEV - 1, N_DEV)
        ge_l = _gelu(comm_l[1].astype(jnp.float32))
        out_ref[pl.ds(own_l * ch, ch), nh:] = ge_l
        comm_l[1] = ge_l.astype(jnp.bfloat16)

        for t in range(N_DEV - 1):
            s = N_DEV - 1 + t
            recv_slot = (s + 1) % 2
            rr = hop(s, comm_r, 0, right)
            rl = hop(s, comm_l, 1, left)
            rr.start()
            rl.start()
            rr.wait()
            c_r = lax.rem(my + 2 * N_DEV - t, N_DEV)
            out_ref[pl.ds(c_r * ch, ch), :nh] = comm_r[recv_slot].astype(jnp.float32)
            rl.wait()
            c_l = lax.rem(my + t, N_DEV)
            out_ref[pl.ds(c_l * ch, ch), nh:] = comm_l[recv_slot].astype(jnp.float32)

    out_shape = jax.ShapeDtypeStruct((m, n), jnp.float32)
    return pl.pallas_call(
        body,
        out_shape=out_shape,
        in_specs=[
            pl.BlockSpec(memory_space=pltpu.VMEM),
            pl.BlockSpec(memory_space=pltpu.VMEM),
        ],
        out_specs=pl.BlockSpec(memory_space=pltpu.VMEM),
        scratch_shapes=[
            pltpu.VMEM((2, ch, nh), jnp.bfloat16),
            pltpu.VMEM((2, ch, nh), jnp.bfloat16),
            pltpu.SemaphoreType.DMA((2, 2, 2)),
        ],
        compiler_params=pltpu.CompilerParams(
            collective_id=0,
            vmem_limit_bytes=60 * 1024 * 1024,
        ),
    )(x, w_mat)


# device time: 248049 ns/iter; 2.8963x vs baseline; 1.0141x over previous
import jax
import jax.numpy as jnp
from jax import lax
from jax.experimental import pallas as pl
from jax.experimental.pallas import tpu as pltpu

N_DEV = 8
N_SUB = 2


def _gelu(y):
    c = 0.7978845608028654
    return 0.5 * y * (1.0 + jnp.tanh(c * (y + 0.044715 * y * y * y)))


def kernel(x, w_mat):
    m, k_sh = x.shape
    _, n = w_mat.shape
    ch = m // N_DEV
    chh = ch // N_SUB
    nh = n // 2
    x = x.astype(jnp.bfloat16)
    w_mat = w_mat.astype(jnp.bfloat16)

    def body(x_ref, w_ref, out_ref, comm_r, comm_l, sems):
        my = lax.axis_index("i")
        left = lax.rem(my + N_DEV - 1, N_DEV)
        right = lax.rem(my + 1, N_DEV)

        barrier_sem = pltpu.get_barrier_semaphore()
        for nbr in (left, right):
            pl.semaphore_signal(
                barrier_sem, inc=1,
                device_id=(nbr,), device_id_type=pl.DeviceIdType.MESH,
            )
        pl.semaphore_wait(barrier_sem, 2)

        def partial_sub(c, half, sub):
            xs = x_ref[pl.ds(c * ch + sub * chh, chh), :]
            ws = w_ref[:, half * nh:(half + 1) * nh]
            return lax.dot_general(
                xs, ws,
                (((1,), (0,)), ((), ())),
                preferred_element_type=jnp.float32,
            )

        def hop(s, comm, direction, sub, dst):
            send_slot = s % 2
            recv_slot = (s + 1) % 2
            return pltpu.make_async_remote_copy(
                src_ref=comm.at[send_slot, pl.ds(sub * chh, chh)],
                dst_ref=comm.at[recv_slot, pl.ds(sub * chh, chh)],
                send_sem=sems.at[direction, sub, 0, send_slot],
                recv_sem=sems.at[direction, sub, 1, recv_slot],
                device_id=(dst,),
                device_id_type=pl.DeviceIdType.MESH,
            )

        def acc(comm, recv_slot, c, half, sub):
            rows = pl.ds(sub * chh, chh)
            comm[recv_slot, rows, :] = (
                comm[recv_slot, rows, :].astype(jnp.float32)
                + partial_sub(c, half, sub)
            ).astype(jnp.bfloat16)

        for sub in range(N_SUB):
            rows = pl.ds(sub * chh, chh)
            comm_r[0, rows, :] = partial_sub(my, 0, sub).astype(jnp.bfloat16)
            comm_l[0, rows, :] = partial_sub(my, 1, sub).astype(jnp.bfloat16)
        for s in range(N_DEV - 1):
            recv_slot = (s + 1) % 2
            c_r = lax.rem(my + 2 * N_DEV - s - 1, N_DEV)
            c_l = lax.rem(my + s + 1, N_DEV)
            rr = [hop(s, comm_r, 0, sub, right) for sub in range(N_SUB)]
            rl = [hop(s, comm_l, 1, sub, left) for sub in range(N_SUB)]
            for rdma in rr + rl:
                rdma.start()
            for sub in range(N_SUB):
                rr[sub].wait()
                acc(comm_r, recv_slot, c_r, 0, sub)
                rl[sub].wait()
                acc(comm_l, recv_slot, c_l, 1, sub)

        own_r = lax.rem(my + 1, N_DEV)
        ge_r = _gelu(comm_r[1].astype(jnp.float32))
        out_ref[pl.ds(own_r * ch, ch), :nh] = ge_r
        comm_r[1] = ge_r.astype(jnp.bfloat16)
        own_l = lax.rem(my + N_DEV - 1, N_DEV)
        ge_l = _gelu(comm_l[1].astype(jnp.float32))
        out_ref[pl.ds(own_l * ch, ch), nh:] = ge_l
        comm_l[1] = ge_l.astype(jnp.bfloat16)

        for t in range(N_DEV - 1):
            s = N_DEV - 1 + t
            recv_slot = (s + 1) % 2
            c_r = lax.rem(my + 2 * N_DEV - t, N_DEV)
            c_l = lax.rem(my + t, N_DEV)
            rr = [hop(s, comm_r, 0, sub, right) for sub in range(N_SUB)]
            rl = [hop(s, comm_l, 1, sub, left) for sub in range(N_SUB)]
            for rdma in rr + rl:
                rdma.start()
            for sub in range(N_SUB):
                rows = pl.ds(sub * chh, chh)
                rr[sub].wait()
                out_ref[pl.ds(c_r * ch + sub * chh, chh), :nh] = comm_r[
                    recv_slot, rows, :
                ].astype(jnp.float32)
                rl[sub].wait()
                out_ref[pl.ds(c_l * ch + sub * chh, chh), nh:] = comm_l[
                    recv_slot, rows, :
                ].astype(jnp.float32)

    out_shape = jax.ShapeDtypeStruct((m, n), jnp.float32)
    return pl.pallas_call(
        body,
        out_shape=out_shape,
        in_specs=[
            pl.BlockSpec(memory_space=pltpu.VMEM),
            pl.BlockSpec(memory_space=pltpu.VMEM),
        ],
        out_specs=pl.BlockSpec(memory_space=pltpu.VMEM),
        scratch_shapes=[
            pltpu.VMEM((2, ch, nh), jnp.bfloat16),
            pltpu.VMEM((2, ch, nh), jnp.bfloat16),
            pltpu.SemaphoreType.DMA((2, N_SUB, 2, 2)),
        ],
        compiler_params=pltpu.CompilerParams(
            collective_id=0,
            vmem_limit_bytes=60 * 1024 * 1024,
        ),
    )(x, w_mat)


# device time: 214208 ns/iter; 3.3539x vs baseline; 1.1580x over previous
import jax
import jax.numpy as jnp
from jax import lax
from jax.experimental import pallas as pl
from jax.experimental.pallas import tpu as pltpu

N_DEV = 8
N_SUB = 2
DEPTH = 3
N_HOPS = 2 * (N_DEV - 1)


def _gelu(y):
    c = 0.7978845608028654
    return 0.5 * y * (1.0 + jnp.tanh(c * (y + 0.044715 * y * y * y)))


def kernel(x, w_mat):
    m, k_sh = x.shape
    _, n = w_mat.shape
    ch = m // N_DEV
    chh = ch // N_SUB
    nh = n // 2
    x = x.astype(jnp.bfloat16)
    w_mat = w_mat.astype(jnp.bfloat16)

    def body(x_ref, w_ref, out_ref, comm_r, comm_l, sems):
        my = lax.axis_index("i")
        left = lax.rem(my + N_DEV - 1, N_DEV)
        right = lax.rem(my + 1, N_DEV)

        barrier_sem = pltpu.get_barrier_semaphore()
        for nbr in (left, right):
            pl.semaphore_signal(
                barrier_sem, inc=1,
                device_id=(nbr,), device_id_type=pl.DeviceIdType.MESH,
            )
        pl.semaphore_wait(barrier_sem, 2)

        def partial_sub(c, half, sub):
            xs = x_ref[pl.ds(c * ch + sub * chh, chh), :]
            ws = w_ref[:, half * nh:(half + 1) * nh]
            return lax.dot_general(
                xs, ws,
                (((1,), (0,)), ((), ())),
                preferred_element_type=jnp.float32,
            )

        comms = (comm_r, comm_l)
        dsts = (right, left)
        descs = {}

        def make(s, direction, sub):
            comm = comms[direction]
            send_slot = s % DEPTH
            recv_slot = (s + 1) % DEPTH
            rows = pl.ds(sub * chh, chh)
            return pltpu.make_async_remote_copy(
                src_ref=comm.at[send_slot, rows],
                dst_ref=comm.at[recv_slot, rows],
                send_sem=sems.at[direction, sub, 0, send_slot],
                recv_sem=sems.at[direction, sub, 1, recv_slot],
                device_id=(dsts[direction],),
                device_id_type=pl.DeviceIdType.MESH,
            )

        def start(s, direction, sub):
            if s >= DEPTH:
                descs[(s - DEPTH, direction, sub)].wait_send()
            d = make(s, direction, sub)
            descs[(s, direction, sub)] = d
            d.start()

        def acc(direction, recv_slot, c, sub):
            comm = comms[direction]
            rows = pl.ds(sub * chh, chh)
            comm[recv_slot, rows, :] = (
                comm[recv_slot, rows, :].astype(jnp.float32)
                + partial_sub(c, direction, sub)
            ).astype(jnp.bfloat16)

        for sub in range(N_SUB):
            rows = pl.ds(sub * chh, chh)
            comm_r[0, rows, :] = partial_sub(my, 0, sub).astype(jnp.bfloat16)
            comm_l[0, rows, :] = partial_sub(my, 1, sub).astype(jnp.bfloat16)
            start(0, 0, sub)
            start(0, 1, sub)
        for s in range(N_DEV - 1):
            recv_slot = (s + 1) % DEPTH
            c_dir = (
                lax.rem(my + 2 * N_DEV - s - 1, N_DEV),
                lax.rem(my + s + 1, N_DEV),
            )
            for sub in range(N_SUB):
                for direction in range(2):
                    descs[(s, direction, sub)].wait_recv()
                    acc(direction, recv_slot, c_dir[direction], sub)
                    if s < N_DEV - 2:
                        start(s + 1, direction, sub)

        red_slot = (N_DEV - 1) % DEPTH
        own = (lax.rem(my + 1, N_DEV), lax.rem(my + N_DEV - 1, N_DEV))
        cols = (slice(None, nh), slice(nh, None))
        for sub in range(N_SUB):
            rows = pl.ds(sub * chh, chh)
            for direction in range(2):
                comm = comms[direction]
                ge = _gelu(comm[red_slot, rows, :].astype(jnp.float32))
                out_ref[pl.ds(own[direction] * ch + sub * chh, chh),
                        cols[direction]] = ge
                comm[red_slot, rows, :] = ge.astype(jnp.bfloat16)
                start(N_DEV - 1, direction, sub)

        for t in range(N_DEV - 1):
            s = N_DEV - 1 + t
            recv_slot = (s + 1) % DEPTH
            c_dir = (
                lax.rem(my + 2 * N_DEV - t, N_DEV),
                lax.rem(my + t, N_DEV),
            )
            for sub in range(N_SUB):
                rows = pl.ds(sub * chh, chh)
                for direction in range(2):
                    descs[(s, direction, sub)].wait_recv()
                    if t < N_DEV - 2:
                        start(s + 1, direction, sub)
                    out_ref[
                        pl.ds(c_dir[direction] * ch + sub * chh, chh),
                        cols[direction],
                    ] = comms[direction][recv_slot, rows, :].astype(jnp.float32)

        for s in range(N_HOPS - DEPTH, N_HOPS):
            for sub in range(N_SUB):
                for direction in range(2):
                    descs[(s, direction, sub)].wait_send()

    out_shape = jax.ShapeDtypeStruct((m, n), jnp.float32)
    return pl.pallas_call(
        body,
        out_shape=out_shape,
        in_specs=[
            pl.BlockSpec(memory_space=pltpu.VMEM),
            pl.BlockSpec(memory_space=pltpu.VMEM),
        ],
        out_specs=pl.BlockSpec(memory_space=pltpu.VMEM),
        scratch_shapes=[
            pltpu.VMEM((DEPTH, ch, nh), jnp.bfloat16),
            pltpu.VMEM((DEPTH, ch, nh), jnp.bfloat16),
            pltpu.SemaphoreType.DMA((2, N_SUB, 2, DEPTH)),
        ],
        compiler_params=pltpu.CompilerParams(
            collective_id=0,
            vmem_limit_bytes=60 * 1024 * 1024,
        ),
    )(x, w_mat)


# device time: 212781 ns/iter; 3.3764x vs baseline; 1.0067x over previous
import jax
import jax.numpy as jnp
from jax import lax
from jax.experimental import pallas as pl
from jax.experimental.pallas import tpu as pltpu

N_DEV = 8
N_SUB = 4
DEPTH = 3
N_HOPS = 2 * (N_DEV - 1)


def _gelu(y):
    c = 0.7978845608028654
    return 0.5 * y * (1.0 + jnp.tanh(c * (y + 0.044715 * y * y * y)))


def kernel(x, w_mat):
    m, k_sh = x.shape
    _, n = w_mat.shape
    ch = m // N_DEV
    chh = ch // N_SUB
    nh = n // 2
    x = x.astype(jnp.bfloat16)
    w_mat = w_mat.astype(jnp.bfloat16)

    def body(x_ref, w_ref, out_ref, comm_r, comm_l, sems):
        my = lax.axis_index("i")
        left = lax.rem(my + N_DEV - 1, N_DEV)
        right = lax.rem(my + 1, N_DEV)

        barrier_sem = pltpu.get_barrier_semaphore()
        for nbr in (left, right):
            pl.semaphore_signal(
                barrier_sem, inc=1,
                device_id=(nbr,), device_id_type=pl.DeviceIdType.MESH,
            )
        pl.semaphore_wait(barrier_sem, 2)

        def partial_sub(c, half, sub):
            xs = x_ref[pl.ds(c * ch + sub * chh, chh), :]
            ws = w_ref[:, half * nh:(half + 1) * nh]
            return lax.dot_general(
                xs, ws,
                (((1,), (0,)), ((), ())),
                preferred_element_type=jnp.float32,
            )

        comms = (comm_r, comm_l)
        dsts = (right, left)
        descs = {}

        def make(s, direction, sub):
            comm = comms[direction]
            send_slot = s % DEPTH
            recv_slot = (s + 1) % DEPTH
            rows = pl.ds(sub * chh, chh)
            return pltpu.make_async_remote_copy(
                src_ref=comm.at[send_slot, rows],
                dst_ref=comm.at[recv_slot, rows],
                send_sem=sems.at[direction, sub, 0, send_slot],
                recv_sem=sems.at[direction, sub, 1, recv_slot],
                device_id=(dsts[direction],),
                device_id_type=pl.DeviceIdType.MESH,
            )

        def start(s, direction, sub):
            if s >= DEPTH:
                descs[(s - DEPTH, direction, sub)].wait_send()
            d = make(s, direction, sub)
            descs[(s, direction, sub)] = d
            d.start()

        def acc(direction, recv_slot, c, sub):
            comm = comms[direction]
            rows = pl.ds(sub * chh, chh)
            comm[recv_slot, rows, :] = (
                comm[recv_slot, rows, :].astype(jnp.float32)
                + partial_sub(c, direction, sub)
            ).astype(jnp.bfloat16)

        for sub in range(N_SUB):
            rows = pl.ds(sub * chh, chh)
            comm_r[0, rows, :] = partial_sub(my, 0, sub).astype(jnp.bfloat16)
            comm_l[0, rows, :] = partial_sub(my, 1, sub).astype(jnp.bfloat16)
            start(0, 0, sub)
            start(0, 1, sub)
        for s in range(N_DEV - 1):
            recv_slot = (s + 1) % DEPTH
            c_dir = (
                lax.rem(my + 2 * N_DEV - s - 1, N_DEV),
                lax.rem(my + s + 1, N_DEV),
            )
            for sub in range(N_SUB):
                for direction in range(2):
                    descs[(s, direction, sub)].wait_recv()
                    acc(direction, recv_slot, c_dir[direction], sub)
                    if s < N_DEV - 2:
                        start(s + 1, direction, sub)

        red_slot = (N_DEV - 1) % DEPTH
        own = (lax.rem(my + 1, N_DEV), lax.rem(my + N_DEV - 1, N_DEV))
        cols = (slice(None, nh), slice(nh, None))
        for sub in range(N_SUB):
            rows = pl.ds(sub * chh, chh)
            for direction in range(2):
                comm = comms[direction]
                ge = _gelu(comm[red_slot, rows, :].astype(jnp.float32))
                out_ref[pl.ds(own[direction] * ch + sub * chh, chh),
                        cols[direction]] = ge
                comm[red_slot, rows, :] = ge.astype(jnp.bfloat16)
                start(N_DEV - 1, direction, sub)

        for t in range(N_DEV - 1):
            s = N_DEV - 1 + t
            recv_slot = (s + 1) % DEPTH
            c_dir = (
                lax.rem(my + 2 * N_DEV - t, N_DEV),
                lax.rem(my + t, N_DEV),
            )
            for sub in range(N_SUB):
                rows = pl.ds(sub * chh, chh)
                for direction in range(2):
                    descs[(s, direction, sub)].wait_recv()
                    if t < N_DEV - 2:
                        start(s + 1, direction, sub)
                    out_ref[
                        pl.ds(c_dir[direction] * ch + sub * chh, chh),
                        cols[direction],
                    ] = comms[direction][recv_slot, rows, :].astype(jnp.float32)

        for s in range(N_HOPS - DEPTH, N_HOPS):
            for sub in range(N_SUB):
                for direction in range(2):
                    descs[(s, direction, sub)].wait_send()

    out_shape = jax.ShapeDtypeStruct((m, n), jnp.float32)
    return pl.pallas_call(
        body,
        out_shape=out_shape,
        in_specs=[
            pl.BlockSpec(memory_space=pltpu.VMEM),
            pl.BlockSpec(memory_space=pltpu.VMEM),
        ],
        out_specs=pl.BlockSpec(memory_space=pltpu.VMEM),
        scratch_shapes=[
            pltpu.VMEM((DEPTH, ch, nh), jnp.bfloat16),
            pltpu.VMEM((DEPTH, ch, nh), jnp.bfloat16),
            pltpu.SemaphoreType.DMA((2, N_SUB, 2, DEPTH)),
        ],
        compiler_params=pltpu.CompilerParams(
            collective_id=0,
            vmem_limit_bytes=60 * 1024 * 1024,
        ),
    )(x, w_mat)
